# baseline (speedup 1.0000x reference)
import functools

import jax
import jax.numpy as jnp
from jax import lax
from jax.experimental import pallas as pl
from jax.experimental.pallas import tpu as pltpu
from jax.experimental.pallas import tpu_sc as plsc

D = 64
ED = 16
PE = 20
G = 256
NLAYER = 3

TILES = 16
CH = 8
BN = 1000
BE = 2048


def _embed_body(xb, tab, out):
    ids = xb[...]
    iota = lax.broadcasted_iota(jnp.int32, (BN, 32), 1)
    oh = (ids == iota).astype(jnp.float32)
    out[...] = jnp.dot(oh, tab[...], preferred_element_type=jnp.float32)


def _edge_body(ea, w, out):
    for q in range(4):
        out[0, q] = jnp.dot(ea[...], w[0, q],
                            preferred_element_type=jnp.float32)


def _mlp_body(h, a0, a1, a2, a3, scale, w1, b1, w2, b2, out):
    hb = h[...]
    t = scale[0, 0] * jnp.dot(hb, w1[...], preferred_element_type=jnp.float32)
    for q, aq in enumerate((a0, a1, a2, a3)):
        t += jnp.dot(aq[0], w1[16 * q:16 * (q + 1), :],
                     preferred_element_type=jnp.float32)
    t = jnp.maximum(t + b1[...], 0.0)
    out[...] = hb + jnp.dot(t, w2[...], preferred_element_type=jnp.float32) + b2[...]


def _pool_body(bid, h, peb, posw, posb, w1, b1, w2, b2, out, acc_h, acc_p):
    b = pl.program_id(0)

    @pl.when(b == 0)
    def _():
        acc_h[...] = jnp.zeros_like(acc_h)
        acc_p[...] = jnp.zeros_like(acc_p)

    gi = lax.broadcasted_iota(jnp.int32, (G, BN), 0)
    oh = (gi == bid[0]).astype(jnp.float32)
    acc_h[...] += jnp.dot(oh, h[...], preferred_element_type=jnp.float32)
    acc_p[...] += jnp.dot(oh, peb[...], preferred_element_type=jnp.float32)

    @pl.when(b == pl.num_programs(0) - 1)
    def _():
        ap = acc_p[...]
        pos = jnp.dot(ap[:, :PE], posw[...], preferred_element_type=jnp.float32)
        pos += ap[:, PE:PE + 1] * posb[...]
        r = jnp.dot(acc_h[...], w1[:D, :], preferred_element_type=jnp.float32)
        r += jnp.dot(pos, w1[D:, :], preferred_element_type=jnp.float32)
        r = jnp.maximum(r + b1[...], 0.0)
        out[...] = jnp.dot(r, w2[...], preferred_element_type=jnp.float32) + b2[...]


def _sc_body(l, nchunk, ipt, rpt, h4, e3, srcg, dstg, zer, out, idxs, idxd, m,
             aggsh, sem):
    c = lax.axis_index("c")
    s = lax.axis_index("s")
    for q in range(2):
        qq = 2 * c + q
        pltpu.sync_copy(zer.at[pl.ds(s * ipt, ipt)],
                        aggsh.at[pl.ds(s * ipt, ipt)])
        plsc.subcore_barrier()

        def chunk(i, carry):
            r0 = s * rpt + i * CH
            pltpu.sync_copy(srcg.at[qq, pl.ds(r0, CH)], idxs)
            pltpu.sync_copy(dstg.at[pl.ds(r0, CH)], idxd)
            pltpu.sync_copy(e3.at[l, qq, pl.ds(r0 * 128, CH * 128)], m)
            cps = [
                pltpu.async_copy(h4.at[idxs.at[j]], m.at[pl.ds(j * 128, 128)],
                                 sem, add=True)
                for j in range(CH)
            ]
            for cp in cps:
                cp.wait()

            @plsc.parallel_loop(0, CH * 128, step=1, unroll=8)
            def relu(r):
                m[r, pl.ds(0, 16)] = jnp.maximum(m[r, pl.ds(0, 16)], 0.0)

            for j in range(CH):
                pltpu.sync_copy(m.at[pl.ds(j * 128, 128)],
                                aggsh.at[idxd.at[j]], add=True)
            return carry

        lax.fori_loop(0, nchunk, chunk, 0)
        plsc.subcore_barrier()
        pltpu.sync_copy(aggsh.at[pl.ds(s * ipt, ipt)],
                        out.at[qq, pl.ds(s * ipt, ipt)])


@functools.lru_cache(maxsize=None)
def _build(n, e):
    rows = -(-e // (128 * TILES * CH)) * (TILES * CH)
    epad = rows * 128
    npad = -(-(n + 1) // 128) * 128
    ipt = npad // TILES
    rpt = rows // TILES
    nchunk = rpt // CH
    nb = n // BN

    embed_call = pl.pallas_call(
        _embed_body,
        grid=(nb,),
        in_specs=[pl.BlockSpec((BN, 1), lambda b: (b, 0)),
                  pl.BlockSpec((32, D), lambda b: (0, 0))],
        out_specs=pl.BlockSpec((BN, D), lambda b: (b, 0)),
        out_shape=jax.ShapeDtypeStruct((n, D), jnp.float32),
    )

    xrows = epad // 8
    edge_call = pl.pallas_call(
        _edge_body,
        grid=(NLAYER, xrows // BE),
        in_specs=[pl.BlockSpec((BE, 8 * (ED + 1)), lambda l, b: (b, 0)),
                  pl.BlockSpec((1, 4, 8 * (ED + 1), 128),
                               lambda l, b: (l, 0, 0, 0))],
        out_specs=pl.BlockSpec((1, 4, BE, 128), lambda l, b: (l, 0, b, 0)),
        out_shape=jax.ShapeDtypeStruct((NLAYER, 4, xrows, 128), jnp.float32),
    )

    mlp_call = pl.pallas_call(
        _mlp_body,
        grid=(nb,),
        in_specs=[pl.BlockSpec((BN, D), lambda b: (b, 0)),
                  pl.BlockSpec((1, BN, 16), lambda b: (0, b, 0)),
                  pl.BlockSpec((1, BN, 16), lambda b: (1, b, 0)),
                  pl.BlockSpec((1, BN, 16), lambda b: (2, b, 0)),
                  pl.BlockSpec((1, BN, 16), lambda b: (3, b, 0)),
                  pl.BlockSpec((1, 1), lambda b: (0, 0)),
                  pl.BlockSpec((D, 2 * D), lambda b: (0, 0)),
                  pl.BlockSpec((1, 2 * D), lambda b: (0, 0)),
                  pl.BlockSpec((2 * D, D), lambda b: (0, 0)),
                  pl.BlockSpec((1, D), lambda b: (0, 0))],
        out_specs=pl.BlockSpec((BN, D), lambda b: (b, 0)),
        out_shape=jax.ShapeDtypeStruct((n, D), jnp.float32),
    )

    dh = D + PE
    pool_call = pl.pallas_call(
        _pool_body,
        grid=(nb,),
        in_specs=[pl.BlockSpec((1, 1, BN), lambda b: (b, 0, 0)),
                  pl.BlockSpec((BN, D), lambda b: (b, 0)),
                  pl.BlockSpec((BN, PE + 1), lambda b: (b, 0)),
                  pl.BlockSpec((PE, PE), lambda b: (0, 0)),
                  pl.BlockSpec((1, PE), lambda b: (0, 0)),
                  pl.BlockSpec((dh, dh // 2), lambda b: (0, 0)),
                  pl.BlockSpec((1, dh // 2), lambda b: (0, 0)),
                  pl.BlockSpec((dh // 2, 1), lambda b: (0, 0)),
                  pl.BlockSpec((1, 1), lambda b: (0, 0))],
        out_specs=pl.BlockSpec((G, 1), lambda b: (0, 0)),
        out_shape=jax.ShapeDtypeStruct((G, 1), jnp.float32),
        scratch_shapes=[pltpu.VMEM((G, D), jnp.float32),
                        pltpu.VMEM((G, PE + 1), jnp.float32)],
    )

    sc_calls = [
        pl.kernel(
            functools.partial(_sc_body, l, nchunk, ipt, rpt),
            out_type=jax.ShapeDtypeStruct((4, npad, 16), jnp.float32),
            mesh=plsc.VectorSubcoreMesh(core_axis_name="c",
                                        subcore_axis_name="s"),
            compiler_params=pltpu.CompilerParams(use_tc_tiling_on_sc=False),
            scratch_types=[pltpu.VMEM((CH, 128), jnp.int32),
                           pltpu.VMEM((CH, 128), jnp.int32),
                           pltpu.VMEM((CH * 128, 16), jnp.float32),
                           pltpu.VMEM_SHARED((npad, 16), jnp.float32),
                           pltpu.SemaphoreType.DMA],
        )
        for l in range(NLAYER)
    ]

    return embed_call, edge_call, mlp_call, pool_call, sc_calls, epad, npad


def kernel(x, edge_index, edge_attr, pe, batch_ids, params):
    f32 = jnp.float32
    n = x.shape[0]
    e = edge_index.shape[1]
    embed_call, edge_call, mlp_call, pool_call, sc_calls, epad, npad = _build(n, e)

    src = edge_index[0].astype(jnp.int32)
    dst = edge_index[1].astype(jnp.int32)
    pad = epad - e
    dump = n
    src_p = jnp.concatenate([src, jnp.zeros((pad,), jnp.int32)])
    dst_p = jnp.concatenate([dst, jnp.full((pad,), dump, jnp.int32)])
    src4 = src_p * 4
    srcg = jnp.stack([src4 + qq for qq in range(4)]).reshape(4, epad // 128, 128)
    dstg = dst_p.reshape(epad // 128, 128)

    ea = jnp.concatenate([edge_attr.astype(f32),
                          jnp.zeros((pad, ED), f32)], axis=0)
    ea8 = jnp.concatenate([ea.reshape(epad // 8, 8, ED),
                           jnp.ones((epad // 8, 8, 1), f32)],
                          axis=2).reshape(epad // 8, 8 * (ED + 1))
    eye8 = jnp.eye(8, dtype=f32)
    w8 = jnp.stack([
        jnp.stack([
            jnp.kron(eye8, jnp.concatenate(
                [lp['edge_W'], lp['edge_b'][None, :]],
                axis=0)[:, 16 * q:16 * (q + 1)])
            for q in range(4)
        ])
        for lp in params['layers']
    ])
    zer = jnp.zeros((npad, 16), f32)
    tab = jnp.concatenate([params['atom_table'],
                           jnp.zeros((32 - 21, D), f32)], axis=0)
    x_i = x.astype(jnp.int32)[:, None]

    h = embed_call(x_i, tab)
    e_sc = edge_call(ea8, w8).reshape(NLAYER, 4, epad, 16)

    for l in range(NLAYER):
        lp = params['layers'][l]
        agg = sc_calls[l](h.reshape(4 * n, 16), e_sc, srcg, dstg, zer)
        scale = (1.0 + lp['eps']).astype(f32).reshape(1, 1)
        h = mlp_call(h, agg, agg, agg, agg, scale, lp['W1'],
                     lp['b1'].reshape(1, 2 * D), lp['W2'],
                     lp['b2'].reshape(1, D))

    pe_aug = jnp.concatenate([pe.astype(f32), jnp.ones((n, 1), f32)], axis=1)
    bid = batch_ids.astype(jnp.int32).reshape(n // BN, 1, BN)
    out = pool_call(bid, h, pe_aug, params['pos_W'],
                    params['pos_b'].reshape(1, PE), params['head_W1'],
                    params['head_b1'].reshape(1, (D + PE) // 2),
                    params['head_W2'], params['head_b2'].reshape(1, 1))
    return out


# speedup vs baseline: 1.0735x; 1.0735x over previous
import functools

import jax
import jax.numpy as jnp
from jax import lax
from jax.experimental import pallas as pl
from jax.experimental.pallas import tpu as pltpu
from jax.experimental.pallas import tpu_sc as plsc

D = 64
ED = 16
PE = 20
G = 256
NLAYER = 3

TILES = 16
CH = 8
BN = 1000
BE = 2048


def _embed_body(xb, tab, out):
    ids = xb[...]
    iota = lax.broadcasted_iota(jnp.int32, (BN, 32), 1)
    oh = (ids == iota).astype(jnp.float32)
    out[...] = jnp.dot(oh, tab[...], preferred_element_type=jnp.float32)


def _edge_body(ea, w, out):
    for q in range(4):
        out[0, q] = jnp.dot(ea[...], w[0, q],
                            preferred_element_type=jnp.float32)


def _mlp_body(h, a0, a1, a2, a3, scale, w1, b1, w2, b2, out):
    hb = h[...]
    t = scale[0, 0] * jnp.dot(hb, w1[...], preferred_element_type=jnp.float32)
    for q, aq in enumerate((a0, a1, a2, a3)):
        t += jnp.dot(aq[0], w1[16 * q:16 * (q + 1), :],
                     preferred_element_type=jnp.float32)
    t = jnp.maximum(t + b1[...], 0.0)
    out[...] = hb + jnp.dot(t, w2[...], preferred_element_type=jnp.float32) + b2[...]


def _pool_body(bid, h, peb, posw, posb, w1, b1, w2, b2, out, acc_h, acc_p):
    b = pl.program_id(0)

    @pl.when(b == 0)
    def _():
        acc_h[...] = jnp.zeros_like(acc_h)
        acc_p[...] = jnp.zeros_like(acc_p)

    gi = lax.broadcasted_iota(jnp.int32, (G, BN), 0)
    oh = (gi == bid[0]).astype(jnp.float32)
    acc_h[...] += jnp.dot(oh, h[...], preferred_element_type=jnp.float32)
    acc_p[...] += jnp.dot(oh, peb[...], preferred_element_type=jnp.float32)

    @pl.when(b == pl.num_programs(0) - 1)
    def _():
        ap = acc_p[...]
        pos = jnp.dot(ap[:, :PE], posw[...], preferred_element_type=jnp.float32)
        pos += ap[:, PE:PE + 1] * posb[...]
        r = jnp.dot(acc_h[...], w1[:D, :], preferred_element_type=jnp.float32)
        r += jnp.dot(pos, w1[D:, :], preferred_element_type=jnp.float32)
        r = jnp.maximum(r + b1[...], 0.0)
        out[...] = jnp.dot(r, w2[...], preferred_element_type=jnp.float32) + b2[...]


def _sc_body(l, nchunk, ipt, rpt, h4, e3, srcg, dstg, zer, out, idxs, idxd, m,
             aggsh, sem):
    c = lax.axis_index("c")
    s = lax.axis_index("s")
    for q in range(2):
        qq = 2 * c + q
        pltpu.sync_copy(zer.at[pl.ds(s * ipt, ipt)],
                        aggsh.at[pl.ds(s * ipt, ipt)])
        plsc.subcore_barrier()

        def chunk(i, carry):
            r0 = s * rpt + i * CH
            pltpu.sync_copy(srcg.at[qq, pl.ds(r0, CH)], idxs)
            pltpu.sync_copy(dstg.at[pl.ds(r0, CH)], idxd)
            pltpu.sync_copy(e3.at[l, qq, pl.ds(r0 * 128, CH * 128)], m)
            cps = [
                pltpu.async_copy(h4.at[idxs.at[j]], m.at[pl.ds(j * 128, 128)],
                                 sem, add=True)
                for j in range(CH)
            ]
            for cp in cps:
                cp.wait()

            @plsc.parallel_loop(0, CH * 128, step=1, unroll=8)
            def relu(r):
                m[r, pl.ds(0, 16)] = jnp.maximum(m[r, pl.ds(0, 16)], 0.0)

            for j in range(CH):
                pltpu.sync_copy(m.at[pl.ds(j * 128, 128)],
                                aggsh.at[idxd.at[j]], add=True)
            return carry

        lax.fori_loop(0, nchunk, chunk, 0)
        plsc.subcore_barrier()
        pltpu.sync_copy(aggsh.at[pl.ds(s * ipt, ipt)],
                        out.at[qq, pl.ds(s * ipt, ipt)])


@functools.lru_cache(maxsize=None)
def _build(n, e):
    rows = -(-e // (128 * TILES * CH)) * (TILES * CH)
    epad = rows * 128
    npad = -(-(n + 1) // 128) * 128
    ipt = npad // TILES
    rpt = rows // TILES
    nchunk = rpt // CH
    nb = n // BN

    embed_call = pl.pallas_call(
        _embed_body,
        grid=(nb,),
        in_specs=[pl.BlockSpec((BN, 1), lambda b: (b, 0)),
                  pl.BlockSpec((32, D), lambda b: (0, 0))],
        out_specs=pl.BlockSpec((BN, D), lambda b: (b, 0)),
        out_shape=jax.ShapeDtypeStruct((n, D), jnp.float32),
    )

    xrows = epad // 8
    edge_call = pl.pallas_call(
        _edge_body,
        grid=(NLAYER, xrows // BE),
        in_specs=[pl.BlockSpec((BE, 8 * (ED + 1)), lambda l, b: (b, 0)),
                  pl.BlockSpec((1, 4, 8 * (ED + 1), 128),
                               lambda l, b: (l, 0, 0, 0))],
        out_specs=pl.BlockSpec((1, 4, BE, 128), lambda l, b: (l, 0, b, 0)),
        out_shape=jax.ShapeDtypeStruct((NLAYER, 4, xrows, 128), jnp.float32),
    )

    mlp_call = pl.pallas_call(
        _mlp_body,
        grid=(nb,),
        in_specs=[pl.BlockSpec((BN, D), lambda b: (b, 0)),
                  pl.BlockSpec((1, BN, 16), lambda b: (0, b, 0)),
                  pl.BlockSpec((1, BN, 16), lambda b: (1, b, 0)),
                  pl.BlockSpec((1, BN, 16), lambda b: (2, b, 0)),
                  pl.BlockSpec((1, BN, 16), lambda b: (3, b, 0)),
                  pl.BlockSpec((1, 1), lambda b: (0, 0)),
                  pl.BlockSpec((D, 2 * D), lambda b: (0, 0)),
                  pl.BlockSpec((1, 2 * D), lambda b: (0, 0)),
                  pl.BlockSpec((2 * D, D), lambda b: (0, 0)),
                  pl.BlockSpec((1, D), lambda b: (0, 0))],
        out_specs=pl.BlockSpec((BN, D), lambda b: (b, 0)),
        out_shape=jax.ShapeDtypeStruct((n, D), jnp.float32),
    )

    dh = D + PE
    pool_call = pl.pallas_call(
        _pool_body,
        grid=(nb,),
        in_specs=[pl.BlockSpec((1, 1, BN), lambda b: (b, 0, 0)),
                  pl.BlockSpec((BN, D), lambda b: (b, 0)),
                  pl.BlockSpec((BN, PE + 1), lambda b: (b, 0)),
                  pl.BlockSpec((PE, PE), lambda b: (0, 0)),
                  pl.BlockSpec((1, PE), lambda b: (0, 0)),
                  pl.BlockSpec((dh, dh // 2), lambda b: (0, 0)),
                  pl.BlockSpec((1, dh // 2), lambda b: (0, 0)),
                  pl.BlockSpec((dh // 2, 1), lambda b: (0, 0)),
                  pl.BlockSpec((1, 1), lambda b: (0, 0))],
        out_specs=pl.BlockSpec((G, 1), lambda b: (0, 0)),
        out_shape=jax.ShapeDtypeStruct((G, 1), jnp.float32),
        scratch_shapes=[pltpu.VMEM((G, D), jnp.float32),
                        pltpu.VMEM((G, PE + 1), jnp.float32)],
    )

    sc_calls = [
        pl.kernel(
            functools.partial(_sc_body, l, nchunk, ipt, rpt),
            out_type=jax.ShapeDtypeStruct((4, npad, 16), jnp.float32),
            mesh=plsc.VectorSubcoreMesh(core_axis_name="c",
                                        subcore_axis_name="s"),
            compiler_params=pltpu.CompilerParams(use_tc_tiling_on_sc=False),
            scratch_types=[pltpu.VMEM((CH, 128), jnp.int32),
                           pltpu.VMEM((CH, 128), jnp.int32),
                           pltpu.VMEM((CH * 128, 16), jnp.float32),
                           pltpu.VMEM_SHARED((npad, 16), jnp.float32),
                           pltpu.SemaphoreType.DMA],
        )
        for l in range(NLAYER)
    ]

    return embed_call, edge_call, mlp_call, pool_call, sc_calls, epad, npad


def kernel(x, edge_index, edge_attr, pe, batch_ids, params):
    f32 = jnp.float32
    n = x.shape[0]
    e = edge_index.shape[1]
    embed_call, edge_call, mlp_call, pool_call, sc_calls, epad, npad = _build(n, e)

    src = edge_index[0].astype(jnp.int32)
    dst = edge_index[1].astype(jnp.int32)
    pad = epad - e
    dump = n
    src_p = jnp.concatenate([src, jnp.zeros((pad,), jnp.int32)])
    dst_p = jnp.concatenate([dst, jnp.full((pad,), dump, jnp.int32)])
    src4 = src_p * 4
    srcg = jnp.stack([src4 + qq for qq in range(4)]).reshape(4, epad // 128, 128)
    dstg = dst_p.reshape(epad // 128, 128)

    ea = jnp.concatenate([edge_attr.astype(f32),
                          jnp.zeros((pad, ED), f32)], axis=0)
    ea_aug = jnp.concatenate([ea, jnp.ones((epad, 1), f32)], axis=1)
    ea8 = ea_aug.reshape(epad // 8, 8 * (ED + 1))
    eye8 = jnp.eye(8, dtype=f32)
    w8 = jnp.stack([
        jnp.stack([
            jnp.kron(eye8, jnp.concatenate(
                [lp['edge_W'], lp['edge_b'][None, :]],
                axis=0)[:, 16 * q:16 * (q + 1)])
            for q in range(4)
        ])
        for lp in params['layers']
    ])
    zer = jnp.zeros((npad, 16), f32)
    tab = jnp.concatenate([params['atom_table'],
                           jnp.zeros((32 - 21, D), f32)], axis=0)
    x_i = x.astype(jnp.int32)[:, None]

    h = embed_call(x_i, tab)
    e_sc = edge_call(ea8, w8).reshape(NLAYER, 4, epad, 16)

    for l in range(NLAYER):
        lp = params['layers'][l]
        agg = sc_calls[l](h.reshape(4 * n, 16), e_sc, srcg, dstg, zer)
        scale = (1.0 + lp['eps']).astype(f32).reshape(1, 1)
        h = mlp_call(h, agg, agg, agg, agg, scale, lp['W1'],
                     lp['b1'].reshape(1, 2 * D), lp['W2'],
                     lp['b2'].reshape(1, D))

    pe_aug = jnp.concatenate([pe.astype(f32), jnp.ones((n, 1), f32)], axis=1)
    bid = batch_ids.astype(jnp.int32).reshape(n // BN, 1, BN)
    out = pool_call(bid, h, pe_aug, params['pos_W'],
                    params['pos_b'].reshape(1, PE), params['head_W1'],
                    params['head_b1'].reshape(1, (D + PE) // 2),
                    params['head_W2'], params['head_b2'].reshape(1, 1))
    return out
